# Initial kernel scaffold; baseline (speedup 1.0000x reference)
import jax
import jax.numpy as jnp
from jax.experimental import pallas as pl


def kernel(x, edge_index, batch, W_pre, b_pre, W_conv, b_conv, gamma, beta, W_post, b_post):
    raise NotImplementedError("write your pallas kernel here")


# speedup vs baseline: 16.9925x; 16.9925x over previous
import functools

import jax
import jax.numpy as jnp
from jax import lax
from jax.experimental import pallas as pl
from jax.experimental.pallas import tpu as pltpu
from jax.experimental.pallas import tpu_sc as plsc

N = 10000
E = 320000
D = 128
H = 128
C = 40
G = 64

NW = 32
NP = 10240
RPT = NP // 16
K = 80
EPW = E // NW
NCH = EPW // K

R = 1000
NB = N // R

_mesh = plsc.VectorSubcoreMesh(core_axis_name="c", subcore_axis_name="s")


@functools.partial(
    pl.kernel,
    out_type=jax.ShapeDtypeStruct((2 * NP,), jnp.float32),
    mesh=_mesh,
    scratch_types=[
        pltpu.VMEM((K,), jnp.int32),
        pltpu.VMEM((K,), jnp.float32),
        pltpu.VMEM((RPT,), jnp.float32),
        pltpu.VMEM_SHARED((NP,), jnp.float32),
    ],
)
def _sc_degree(dst_hbm, out_hbm, dst_v, ones_v, zb, hist):
    cid = lax.axis_index("c")
    sid = lax.axis_index("s")
    wid = sid * 2 + cid

    z16 = jnp.zeros((16,), jnp.float32)
    o16 = jnp.ones((16,), jnp.float32)

    def zb_body(j, carry):
        zb[pl.ds(j * 16, 16)] = z16
        return carry

    lax.fori_loop(0, RPT // 16, zb_body, 0)
    for j in range(K // 16):
        ones_v[pl.ds(j * 16, 16)] = o16

    pltpu.sync_copy(zb, hist.at[pl.ds(sid * RPT, RPT)])
    plsc.subcore_barrier()

    base_w = wid * EPW

    def body(i, carry):
        base = pl.multiple_of(base_w + i * K, 8)
        pltpu.sync_copy(dst_hbm.at[pl.ds(base, K)], dst_v)
        pltpu.sync_copy(ones_v, hist.at[dst_v], add=True)
        return carry

    lax.fori_loop(0, NCH, body, 0)

    plsc.subcore_barrier()
    off = pl.multiple_of(cid * NP + sid * RPT, 8)
    pltpu.sync_copy(hist.at[pl.ds(sid * RPT, RPT)], out_hbm.at[pl.ds(off, RPT)])


@functools.partial(
    pl.kernel,
    out_type=jax.ShapeDtypeStruct((2 * NP, H), jnp.float32),
    mesh=_mesh,
    scratch_types=[
        pltpu.VMEM((K,), jnp.int32),
        pltpu.VMEM((K,), jnp.int32),
        pltpu.VMEM((K, H), jnp.float32),
        pltpu.VMEM((128, H), jnp.float32),
        pltpu.VMEM_SHARED((NP, H), jnp.float32),
        pltpu.SemaphoreType.DMA,
    ],
)
def _sc_edge_agg(src_hbm, dst_hbm, hs_hbm, out_hbm, src_v, dst_v, rows_v, zbuf,
                 acc, sem):
    cid = lax.axis_index("c")
    sid = lax.axis_index("s")
    wid = sid * 2 + cid

    z16 = jnp.zeros((16,), jnp.float32)

    def zb_body(j, carry):
        r = j // (H // 16)
        c = j - r * (H // 16)
        zbuf[r, pl.ds(c * 16, 16)] = z16
        return carry

    lax.fori_loop(0, 128 * (H // 16), zb_body, 0)

    for j in range(RPT // 128):
        pltpu.sync_copy(zbuf, acc.at[pl.ds(sid * RPT + j * 128, 128)])
    plsc.subcore_barrier()

    base_w = wid * EPW

    def body(i, carry):
        base = pl.multiple_of(base_w + i * K, 8)
        pltpu.sync_copy(src_hbm.at[pl.ds(base, K)], src_v)
        pltpu.sync_copy(dst_hbm.at[pl.ds(base, K)], dst_v)
        pltpu.async_copy(hs_hbm.at[src_v], rows_v, sem).wait()
        pltpu.sync_copy(rows_v, acc.at[dst_v], add=True)
        return carry

    lax.fori_loop(0, NCH, body, 0)

    plsc.subcore_barrier()
    pltpu.sync_copy(acc.at[pl.ds(sid * RPT, RPT)],
                    out_hbm.at[pl.ds(cid * NP + sid * RPT, RPT)])


def _tc_pre_body(x_ref, wpre_ref, wconv_ref, bpre_ref, out_ref, wb, bb):
    i = pl.program_id(0)

    @pl.when(i == 0)
    def _():
        wb[...] = jnp.dot(wpre_ref[...], wconv_ref[...],
                          preferred_element_type=jnp.float32)
        bb[...] = jnp.dot(bpre_ref[...], wconv_ref[...],
                          preferred_element_type=jnp.float32)

    out_ref[...] = jnp.dot(x_ref[...], wb[...],
                           preferred_element_type=jnp.float32) + bb[...]


def _tc_pre(x, W_pre, b_pre, W_conv):
    return pl.pallas_call(
        _tc_pre_body,
        grid=(NB,),
        in_specs=[
            pl.BlockSpec((R, D), lambda i: (i, 0)),
            pl.BlockSpec((D, H), lambda i: (0, 0)),
            pl.BlockSpec((D, H), lambda i: (0, 0)),
            pl.BlockSpec((1, D), lambda i: (0, 0)),
        ],
        out_specs=pl.BlockSpec((R, H), lambda i: (i, 0)),
        out_shape=jax.ShapeDtypeStruct((N, H), jnp.float32),
        scratch_shapes=[
            pltpu.VMEM((D, H), jnp.float32),
            pltpu.VMEM((1, H), jnp.float32),
        ],
    )(x, W_pre, W_conv, b_pre.reshape(1, D))


def _tc_scale_body(h2_ref, d0_ref, d1_ref, hs_ref, dinv_ref):
    deg = d0_ref[...] + d1_ref[...] + 1.0
    dinv = lax.rsqrt(jnp.maximum(deg, 1e-12))
    dinv_ref[...] = dinv
    hs_ref[...] = h2_ref[...] * dinv


def _tc_scale(h2, deg0, deg1):
    return pl.pallas_call(
        _tc_scale_body,
        grid=(NB,),
        in_specs=[
            pl.BlockSpec((R, H), lambda i: (i, 0)),
            pl.BlockSpec((R, 1), lambda i: (i, 0)),
            pl.BlockSpec((R, 1), lambda i: (i, 0)),
        ],
        out_specs=[
            pl.BlockSpec((R, H), lambda i: (i, 0)),
            pl.BlockSpec((R, 1), lambda i: (i, 0)),
        ],
        out_shape=[
            jax.ShapeDtypeStruct((N, H), jnp.float32),
            jax.ShapeDtypeStruct((N, 1), jnp.float32),
        ],
    )(h2, deg0, deg1)


def _tc_stats_body(p0_ref, p1_ref, hs_ref, dinv_ref, bconv_ref,
                   t_ref, sums_ref, acc):
    i = pl.program_id(0)

    @pl.when(i == 0)
    def _():
        acc[...] = jnp.zeros_like(acc)

    t = dinv_ref[...] * (p0_ref[...] + p1_ref[...] + hs_ref[...]) + bconv_ref[...]
    t_ref[...] = t
    acc[0:1, :] += jnp.sum(t, axis=0, keepdims=True)
    acc[1:2, :] += jnp.sum(t * t, axis=0, keepdims=True)

    @pl.when(i == NB - 1)
    def _():
        sums_ref[...] = acc[...]


def _tc_stats(p0, p1, hs, dinv, b_conv):
    return pl.pallas_call(
        _tc_stats_body,
        grid=(NB,),
        in_specs=[
            pl.BlockSpec((R, H), lambda i: (i, 0)),
            pl.BlockSpec((R, H), lambda i: (i, 0)),
            pl.BlockSpec((R, H), lambda i: (i, 0)),
            pl.BlockSpec((R, 1), lambda i: (i, 0)),
            pl.BlockSpec((1, H), lambda i: (0, 0)),
        ],
        out_specs=[
            pl.BlockSpec((R, H), lambda i: (i, 0)),
            pl.BlockSpec((2, H), lambda i: (0, 0)),
        ],
        out_shape=[
            jax.ShapeDtypeStruct((N, H), jnp.float32),
            jax.ShapeDtypeStruct((2, H), jnp.float32),
        ],
        scratch_shapes=[pltpu.VMEM((2, H), jnp.float32)],
    )(p0, p1, hs, dinv, b_conv.reshape(1, H))


def _tc_final_body(t_ref, sums_ref, gamma_ref, beta_ref, batch_ref,
                   wpost_ref, bpost_ref, out_ref, racc):
    i = pl.program_id(0)

    @pl.when(i == 0)
    def _():
        racc[...] = jnp.zeros_like(racc)

    mean = sums_ref[0:1, :] / N
    msq = sums_ref[1:2, :] / N
    var = msq - mean * mean
    inv = lax.rsqrt(var + 1e-5)
    scale = gamma_ref[...] * inv
    shift = beta_ref[...] - mean * scale

    ha = jnp.maximum(t_ref[...] * scale + shift, 0.0)
    gids = lax.broadcasted_iota(jnp.int32, (R, G), 1)
    onehot = jnp.where(batch_ref[...] == gids, 1.0, 0.0)
    racc[...] += lax.dot_general(onehot, ha, (((0,), (0,)), ((), ())),
                                 preferred_element_type=jnp.float32)

    @pl.when(i == NB - 1)
    def _():
        out_ref[...] = jnp.dot(racc[...], wpost_ref[...],
                               preferred_element_type=jnp.float32) + bpost_ref[...]


def _tc_final(t, sums, gamma, beta, batch2d, W_post_pad, b_post_pad):
    return pl.pallas_call(
        _tc_final_body,
        grid=(NB,),
        in_specs=[
            pl.BlockSpec((R, H), lambda i: (i, 0)),
            pl.BlockSpec((2, H), lambda i: (0, 0)),
            pl.BlockSpec((1, H), lambda i: (0, 0)),
            pl.BlockSpec((1, H), lambda i: (0, 0)),
            pl.BlockSpec((R, 1), lambda i: (i, 0)),
            pl.BlockSpec((H, 128), lambda i: (0, 0)),
            pl.BlockSpec((1, 128), lambda i: (0, 0)),
        ],
        out_specs=pl.BlockSpec((G, 128), lambda i: (0, 0)),
        out_shape=jax.ShapeDtypeStruct((G, 128), jnp.float32),
        scratch_shapes=[pltpu.VMEM((G, H), jnp.float32)],
    )(t, sums, gamma.reshape(1, H), beta.reshape(1, H), batch2d,
      W_post_pad, b_post_pad)


def kernel(x, edge_index, batch, W_pre, b_pre, W_conv, b_conv, gamma, beta,
           W_post, b_post):
    src = edge_index[0]
    dst = edge_index[1]

    hist = _sc_degree(dst)
    h2 = _tc_pre(x, W_pre, b_pre, W_conv)

    deg0 = hist[:N].reshape(N, 1)
    deg1 = hist[NP:NP + N].reshape(N, 1)
    hs, dinv = _tc_scale(h2, deg0, deg1)

    aggp = _sc_edge_agg(src, dst, hs)
    p0 = aggp[:N]
    p1 = aggp[NP:NP + N]

    t, sums = _tc_stats(p0, p1, hs, dinv, b_conv)

    W_post_pad = jnp.pad(W_post, ((0, 0), (0, 128 - C)))
    b_post_pad = jnp.pad(b_post, (0, 128 - C)).reshape(1, 128)
    out_pad = _tc_final(t, sums, gamma, beta, batch.reshape(N, 1),
                        W_post_pad, b_post_pad)
    return out_pad[:, :C]


# speedup vs baseline: 27.1224x; 1.5961x over previous
import functools

import jax
import jax.numpy as jnp
from jax import lax
from jax.experimental import pallas as pl
from jax.experimental.pallas import tpu as pltpu
from jax.experimental.pallas import tpu_sc as plsc

N = 10000
E = 320000
D = 128
H = 128
C = 40
G = 64

NW = 32
NP = 10240
RPT = NP // 16
K = 80
EPW = E // NW
NCH = EPW // K

R = 1000
NB = N // R

_mesh = plsc.VectorSubcoreMesh(core_axis_name="c", subcore_axis_name="s")


@functools.partial(
    pl.kernel,
    out_type=jax.ShapeDtypeStruct((2 * NP,), jnp.float32),
    mesh=_mesh,
    scratch_types=[
        pltpu.VMEM((K,), jnp.int32),
        pltpu.VMEM((K,), jnp.float32),
        pltpu.VMEM((RPT,), jnp.float32),
        pltpu.VMEM_SHARED((NP,), jnp.float32),
    ],
)
def _sc_degree(dst_hbm, out_hbm, dst_v, ones_v, zb, hist):
    cid = lax.axis_index("c")
    sid = lax.axis_index("s")
    wid = sid * 2 + cid

    z16 = jnp.zeros((16,), jnp.float32)
    o16 = jnp.ones((16,), jnp.float32)

    def zb_body(j, carry):
        zb[pl.ds(j * 16, 16)] = z16
        return carry

    lax.fori_loop(0, RPT // 16, zb_body, 0)
    for j in range(K // 16):
        ones_v[pl.ds(j * 16, 16)] = o16

    pltpu.sync_copy(zb, hist.at[pl.ds(sid * RPT, RPT)])
    plsc.subcore_barrier()

    base_w = wid * EPW

    def body(i, carry):
        base = pl.multiple_of(base_w + i * K, 8)
        pltpu.sync_copy(dst_hbm.at[pl.ds(base, K)], dst_v)
        pltpu.sync_copy(ones_v, hist.at[dst_v], add=True)
        return carry

    lax.fori_loop(0, NCH, body, 0)

    plsc.subcore_barrier()
    off = pl.multiple_of(cid * NP + sid * RPT, 8)
    pltpu.sync_copy(hist.at[pl.ds(sid * RPT, RPT)], out_hbm.at[pl.ds(off, RPT)])


OSPAN = 624
ZR = 128


@functools.partial(
    pl.kernel,
    out_type=jax.ShapeDtypeStruct((2 * N, H), jnp.float32),
    mesh=_mesh,
    scratch_types=[
        pltpu.VMEM((2, K), jnp.int32),
        pltpu.VMEM((2, K), jnp.int32),
        pltpu.VMEM((K, H), jnp.float32),
        pltpu.VMEM((K, H), jnp.float32),
        pltpu.VMEM((ZR, H), jnp.float32),
        pltpu.VMEM_SHARED((NP, H), jnp.float32),
        pltpu.SemaphoreType.DMA,
        pltpu.SemaphoreType.DMA,
        pltpu.SemaphoreType.DMA,
        pltpu.SemaphoreType.DMA,
    ],
)
def _sc_edge_agg(src_hbm, dst_hbm, hs_hbm, out_hbm, i2a, i2b, ra, rb, zbuf,
                 acc, sga, sgb, sia, sib):
    cid = lax.axis_index("c")
    sid = lax.axis_index("s")
    wid = sid * 2 + cid

    z16 = jnp.zeros((16,), jnp.float32)

    def zb_body(j, carry):
        r = j // (H // 16)
        c = j - r * (H // 16)
        zbuf[r, pl.ds(c * 16, 16)] = z16
        return carry

    lax.fori_loop(0, ZR * (H // 16), zb_body, 0)

    for j in range(RPT // ZR):
        pltpu.sync_copy(zbuf, acc.at[pl.ds(sid * RPT + j * ZR, ZR)])
    plsc.subcore_barrier()

    base_w = wid * EPW

    def start_chunk(i, i2, rv, sem, sem_i):
        base = pl.multiple_of(base_w + i * K, 8)
        ca = pltpu.async_copy(src_hbm.at[pl.ds(base, K)], i2.at[0], sem_i)
        cb = pltpu.async_copy(dst_hbm.at[pl.ds(base, K)], i2.at[1], sem_i)
        ca.wait()
        cb.wait()
        pltpu.async_copy(hs_hbm.at[i2.at[0]], rv, sem)

    def fin_chunk(i2, rv, sem):
        pltpu.make_async_copy(hs_hbm.at[i2.at[0]], rv, sem).wait()
        pltpu.sync_copy(rv, acc.at[i2.at[1]], add=True)

    start_chunk(0, i2a, ra, sga, sia)

    def body(p, carry):
        a = p * 2
        b = a + 1

        @pl.when(b < NCH)
        def _():
            start_chunk(b, i2b, rb, sgb, sib)

        fin_chunk(i2a, ra, sga)

        @pl.when(b < NCH)
        def _():
            @pl.when(b + 1 < NCH)
            def _():
                start_chunk(b + 1, i2a, ra, sga, sia)

            fin_chunk(i2b, rb, sgb)

        return carry

    lax.fori_loop(0, (NCH + 1) // 2, body, 0)

    plsc.subcore_barrier()

    @pl.when(sid < 15)
    def _():
        pltpu.sync_copy(acc.at[pl.ds(sid * OSPAN, OSPAN)],
                        out_hbm.at[pl.ds(cid * N + sid * OSPAN, OSPAN)])

    @pl.when(sid == 15)
    def _():
        pltpu.sync_copy(acc.at[pl.ds(15 * OSPAN, N - 15 * OSPAN)],
                        out_hbm.at[pl.ds(cid * N + 15 * OSPAN, N - 15 * OSPAN)])


def _tc_pre_body(x_ref, wpre_ref, wconv_ref, bpre_ref, out_ref, wb, bb):
    i = pl.program_id(0)

    @pl.when(i == 0)
    def _():
        wb[...] = jnp.dot(wpre_ref[...], wconv_ref[...],
                          preferred_element_type=jnp.float32)
        bb[...] = jnp.dot(bpre_ref[...], wconv_ref[...],
                          preferred_element_type=jnp.float32)

    out_ref[...] = jnp.dot(x_ref[...], wb[...],
                           preferred_element_type=jnp.float32) + bb[...]


def _tc_pre(x, W_pre, b_pre, W_conv):
    return pl.pallas_call(
        _tc_pre_body,
        grid=(NB,),
        in_specs=[
            pl.BlockSpec((R, D), lambda i: (i, 0)),
            pl.BlockSpec((D, H), lambda i: (0, 0)),
            pl.BlockSpec((D, H), lambda i: (0, 0)),
            pl.BlockSpec((1, D), lambda i: (0, 0)),
        ],
        out_specs=pl.BlockSpec((R, H), lambda i: (i, 0)),
        out_shape=jax.ShapeDtypeStruct((N, H), jnp.float32),
        scratch_shapes=[
            pltpu.VMEM((D, H), jnp.float32),
            pltpu.VMEM((1, H), jnp.float32),
        ],
    )(x, W_pre, W_conv, b_pre.reshape(1, D))


def _tc_scale_body(h2_ref, d0_ref, d1_ref, hs_ref, dinv_ref):
    deg = d0_ref[...] + d1_ref[...] + 1.0
    dinv = lax.rsqrt(jnp.maximum(deg, 1e-12))
    dinv_ref[...] = dinv
    hs_ref[...] = h2_ref[...] * dinv


def _tc_scale(h2, deg0, deg1):
    return pl.pallas_call(
        _tc_scale_body,
        grid=(NB,),
        in_specs=[
            pl.BlockSpec((R, H), lambda i: (i, 0)),
            pl.BlockSpec((R, 1), lambda i: (i, 0)),
            pl.BlockSpec((R, 1), lambda i: (i, 0)),
        ],
        out_specs=[
            pl.BlockSpec((R, H), lambda i: (i, 0)),
            pl.BlockSpec((R, 1), lambda i: (i, 0)),
        ],
        out_shape=[
            jax.ShapeDtypeStruct((N, H), jnp.float32),
            jax.ShapeDtypeStruct((N, 1), jnp.float32),
        ],
    )(h2, deg0, deg1)


def _tc_stats_body(p0_ref, p1_ref, hs_ref, dinv_ref, bconv_ref,
                   t_ref, sums_ref, acc):
    i = pl.program_id(0)

    @pl.when(i == 0)
    def _():
        acc[...] = jnp.zeros_like(acc)

    t = dinv_ref[...] * (p0_ref[...] + p1_ref[...] + hs_ref[...]) + bconv_ref[...]
    t_ref[...] = t
    acc[0:1, :] += jnp.sum(t, axis=0, keepdims=True)
    acc[1:2, :] += jnp.sum(t * t, axis=0, keepdims=True)

    @pl.when(i == NB - 1)
    def _():
        sums_ref[...] = acc[...]


def _tc_stats(aggp, hs, dinv, b_conv):
    return pl.pallas_call(
        _tc_stats_body,
        grid=(NB,),
        in_specs=[
            pl.BlockSpec((R, H), lambda i: (i, 0)),
            pl.BlockSpec((R, H), lambda i: (i + NB, 0)),
            pl.BlockSpec((R, H), lambda i: (i, 0)),
            pl.BlockSpec((R, 1), lambda i: (i, 0)),
            pl.BlockSpec((1, H), lambda i: (0, 0)),
        ],
        out_specs=[
            pl.BlockSpec((R, H), lambda i: (i, 0)),
            pl.BlockSpec((2, H), lambda i: (0, 0)),
        ],
        out_shape=[
            jax.ShapeDtypeStruct((N, H), jnp.float32),
            jax.ShapeDtypeStruct((2, H), jnp.float32),
        ],
        scratch_shapes=[pltpu.VMEM((2, H), jnp.float32)],
    )(aggp, aggp, hs, dinv, b_conv.reshape(1, H))


def _tc_final_body(t_ref, sums_ref, gamma_ref, beta_ref, batch_ref,
                   wpost_ref, bpost_ref, out_ref, racc):
    i = pl.program_id(0)

    @pl.when(i == 0)
    def _():
        racc[...] = jnp.zeros_like(racc)

    mean = sums_ref[0:1, :] / N
    msq = sums_ref[1:2, :] / N
    var = msq - mean * mean
    inv = lax.rsqrt(var + 1e-5)
    scale = gamma_ref[...] * inv
    shift = beta_ref[...] - mean * scale

    ha = jnp.maximum(t_ref[...] * scale + shift, 0.0)
    gids = lax.broadcasted_iota(jnp.int32, (R, G), 1)
    onehot = jnp.where(batch_ref[...] == gids, 1.0, 0.0)
    racc[...] += lax.dot_general(onehot, ha, (((0,), (0,)), ((), ())),
                                 preferred_element_type=jnp.float32)

    @pl.when(i == NB - 1)
    def _():
        out_ref[...] = jnp.dot(racc[...], wpost_ref[...],
                               preferred_element_type=jnp.float32) + bpost_ref[...]


def _tc_final(t, sums, gamma, beta, batch2d, W_post_pad, b_post_pad):
    return pl.pallas_call(
        _tc_final_body,
        grid=(NB,),
        in_specs=[
            pl.BlockSpec((R, H), lambda i: (i, 0)),
            pl.BlockSpec((2, H), lambda i: (0, 0)),
            pl.BlockSpec((1, H), lambda i: (0, 0)),
            pl.BlockSpec((1, H), lambda i: (0, 0)),
            pl.BlockSpec((R, 1), lambda i: (i, 0)),
            pl.BlockSpec((H, 128), lambda i: (0, 0)),
            pl.BlockSpec((1, 128), lambda i: (0, 0)),
        ],
        out_specs=pl.BlockSpec((G, 128), lambda i: (0, 0)),
        out_shape=jax.ShapeDtypeStruct((G, 128), jnp.float32),
        scratch_shapes=[pltpu.VMEM((G, H), jnp.float32)],
    )(t, sums, gamma.reshape(1, H), beta.reshape(1, H), batch2d,
      W_post_pad, b_post_pad)


def kernel(x, edge_index, batch, W_pre, b_pre, W_conv, b_conv, gamma, beta,
           W_post, b_post):
    src = edge_index[0]
    dst = edge_index[1]

    hist = _sc_degree(dst)
    h2 = _tc_pre(x, W_pre, b_pre, W_conv)

    deg0 = hist[:N].reshape(N, 1)
    deg1 = hist[NP:NP + N].reshape(N, 1)
    hs, dinv = _tc_scale(h2, deg0, deg1)

    aggp = _sc_edge_agg(src, dst, hs)

    t, sums = _tc_stats(aggp, hs, dinv, b_conv)

    W_post_pad = jnp.pad(W_post, ((0, 0), (0, 128 - C)))
    b_post_pad = jnp.pad(b_post, (0, 128 - C)).reshape(1, 128)
    out_pad = _tc_final(t, sums, gamma, beta, batch.reshape(N, 1),
                        W_post_pad, b_post_pad)
    return out_pad[:, :C]


# speedup vs baseline: 30.7121x; 1.1324x over previous
import functools

import jax
import jax.numpy as jnp
from jax import lax
from jax.experimental import pallas as pl
from jax.experimental.pallas import tpu as pltpu
from jax.experimental.pallas import tpu_sc as plsc

N = 10000
E = 320000
D = 128
H = 128
C = 40
G = 64

NW = 32
NP = 10240
RPT = NP // 16
K = 80
EPW = E // NW
NCH = EPW // K

R = 1000
NB = N // R

_mesh = plsc.VectorSubcoreMesh(core_axis_name="c", subcore_axis_name="s")


@functools.partial(
    pl.kernel,
    out_type=jax.ShapeDtypeStruct((2 * NP,), jnp.float32),
    mesh=_mesh,
    scratch_types=[
        pltpu.VMEM((K,), jnp.int32),
        pltpu.VMEM((K,), jnp.int32),
        pltpu.VMEM((K,), jnp.float32),
        pltpu.VMEM((RPT,), jnp.float32),
        pltpu.VMEM_SHARED((NP,), jnp.float32),
        pltpu.SemaphoreType.DMA,
        pltpu.SemaphoreType.DMA,
    ],
)
def _sc_degree(dst_hbm, out_hbm, da, db, ones_v, zb, hist, sa, sb):
    cid = lax.axis_index("c")
    sid = lax.axis_index("s")
    wid = sid * 2 + cid

    z16 = jnp.zeros((16,), jnp.float32)
    o16 = jnp.ones((16,), jnp.float32)

    def zb_body(j, carry):
        zb[pl.ds(j * 16, 16)] = z16
        return carry

    lax.fori_loop(0, RPT // 16, zb_body, 0)
    for j in range(K // 16):
        ones_v[pl.ds(j * 16, 16)] = o16

    pltpu.sync_copy(zb, hist.at[pl.ds(sid * RPT, RPT)])
    plsc.subcore_barrier()

    base_w = wid * EPW

    def start_chunk(i, d_v, sem):
        base = pl.multiple_of(base_w + i * K, 8)
        pltpu.async_copy(dst_hbm.at[pl.ds(base, K)], d_v, sem)

    def fin_chunk(d_v, sem):
        pltpu.make_async_copy(dst_hbm.at[pl.ds(0, K)], d_v, sem).wait()
        pltpu.sync_copy(ones_v, hist.at[d_v], add=True)

    start_chunk(0, da, sa)

    def body(p, carry):
        b = p * 2 + 1

        @pl.when(b < NCH)
        def _():
            start_chunk(b, db, sb)

        fin_chunk(da, sa)

        @pl.when(b < NCH)
        def _():
            @pl.when(b + 1 < NCH)
            def _():
                start_chunk(b + 1, da, sa)

            fin_chunk(db, sb)

        return carry

    lax.fori_loop(0, (NCH + 1) // 2, body, 0)

    plsc.subcore_barrier()
    off = pl.multiple_of(cid * NP + sid * RPT, 8)
    pltpu.sync_copy(hist.at[pl.ds(sid * RPT, RPT)], out_hbm.at[pl.ds(off, RPT)])


OSPAN = 624
ZR = 128


@functools.partial(
    pl.kernel,
    out_type=jax.ShapeDtypeStruct((2 * N, H), jnp.float32),
    mesh=_mesh,
    scratch_types=[
        pltpu.VMEM((2, K), jnp.int32),
        pltpu.VMEM((2, K), jnp.int32),
        pltpu.VMEM((K, H), jnp.float32),
        pltpu.VMEM((K, H), jnp.float32),
        pltpu.VMEM((ZR, H), jnp.float32),
        pltpu.VMEM_SHARED((NP, H), jnp.float32),
        pltpu.SemaphoreType.DMA,
        pltpu.SemaphoreType.DMA,
        pltpu.SemaphoreType.DMA,
        pltpu.SemaphoreType.DMA,
    ],
)
def _sc_edge_agg(src_hbm, dst_hbm, hs_hbm, out_hbm, i2a, i2b, ra, rb, zbuf,
                 acc, sga, sgb, sia, sib):
    cid = lax.axis_index("c")
    sid = lax.axis_index("s")
    wid = sid * 2 + cid

    z16 = jnp.zeros((16,), jnp.float32)

    def zb_body(j, carry):
        r = j // (H // 16)
        c = j - r * (H // 16)
        zbuf[r, pl.ds(c * 16, 16)] = z16
        return carry

    lax.fori_loop(0, ZR * (H // 16), zb_body, 0)

    for j in range(RPT // ZR):
        pltpu.sync_copy(zbuf, acc.at[pl.ds(sid * RPT + j * ZR, ZR)])
    plsc.subcore_barrier()

    base_w = wid * EPW

    def start_chunk(i, i2, rv, sem, sem_i):
        base = pl.multiple_of(base_w + i * K, 8)
        ca = pltpu.async_copy(src_hbm.at[pl.ds(base, K)], i2.at[0], sem_i)
        cb = pltpu.async_copy(dst_hbm.at[pl.ds(base, K)], i2.at[1], sem_i)
        ca.wait()
        cb.wait()
        pltpu.async_copy(hs_hbm.at[i2.at[0]], rv, sem)

    def fin_chunk(i2, rv, sem):
        pltpu.make_async_copy(hs_hbm.at[i2.at[0]], rv, sem).wait()
        pltpu.sync_copy(rv, acc.at[i2.at[1]], add=True)

    start_chunk(0, i2a, ra, sga, sia)

    def body(p, carry):
        a = p * 2
        b = a + 1

        @pl.when(b < NCH)
        def _():
            start_chunk(b, i2b, rb, sgb, sib)

        fin_chunk(i2a, ra, sga)

        @pl.when(b < NCH)
        def _():
            @pl.when(b + 1 < NCH)
            def _():
                start_chunk(b + 1, i2a, ra, sga, sia)

            fin_chunk(i2b, rb, sgb)

        return carry

    lax.fori_loop(0, (NCH + 1) // 2, body, 0)

    plsc.subcore_barrier()

    @pl.when(sid < 15)
    def _():
        pltpu.sync_copy(acc.at[pl.ds(sid * OSPAN, OSPAN)],
                        out_hbm.at[pl.ds(cid * N + sid * OSPAN, OSPAN)])

    @pl.when(sid == 15)
    def _():
        pltpu.sync_copy(acc.at[pl.ds(15 * OSPAN, N - 15 * OSPAN)],
                        out_hbm.at[pl.ds(cid * N + 15 * OSPAN, N - 15 * OSPAN)])


def _tc_head_body(x_ref, wpre_ref, wconv_ref, bpre_ref, d0_ref, d1_ref,
                  hs_ref, dinv_ref, wb, bb):
    i = pl.program_id(0)

    @pl.when(i == 0)
    def _():
        wb[...] = jnp.dot(wpre_ref[...], wconv_ref[...],
                          preferred_element_type=jnp.float32)
        bb[...] = jnp.dot(bpre_ref[...], wconv_ref[...],
                          preferred_element_type=jnp.float32)

    h2 = jnp.dot(x_ref[...], wb[...],
                 preferred_element_type=jnp.float32) + bb[...]
    deg = d0_ref[...] + d1_ref[...] + 1.0
    dinv = lax.rsqrt(jnp.maximum(deg, 1e-12))
    dinv_ref[...] = dinv
    hs_ref[...] = h2 * dinv


def _tc_head(x, W_pre, b_pre, W_conv, deg0, deg1):
    return pl.pallas_call(
        _tc_head_body,
        grid=(NB,),
        in_specs=[
            pl.BlockSpec((R, D), lambda i: (i, 0)),
            pl.BlockSpec((D, H), lambda i: (0, 0)),
            pl.BlockSpec((D, H), lambda i: (0, 0)),
            pl.BlockSpec((1, D), lambda i: (0, 0)),
            pl.BlockSpec((R, 1), lambda i: (i, 0)),
            pl.BlockSpec((R, 1), lambda i: (i, 0)),
        ],
        out_specs=[
            pl.BlockSpec((R, H), lambda i: (i, 0)),
            pl.BlockSpec((R, 1), lambda i: (i, 0)),
        ],
        out_shape=[
            jax.ShapeDtypeStruct((N, H), jnp.float32),
            jax.ShapeDtypeStruct((N, 1), jnp.float32),
        ],
        scratch_shapes=[
            pltpu.VMEM((D, H), jnp.float32),
            pltpu.VMEM((1, H), jnp.float32),
        ],
    )(x, W_pre, W_conv, b_pre.reshape(1, D), deg0, deg1)


def _tc_tail_body(p0_ref, p1_ref, hs_ref, dinv_ref, bconv_ref, gamma_ref,
                  beta_ref, batch_ref, wpost_ref, bpost_ref, out_ref,
                  tbuf, acc, racc):
    p = pl.program_id(0)
    i = pl.program_id(1)

    @pl.when((p == 0) & (i == 0))
    def _():
        acc[...] = jnp.zeros_like(acc)
        racc[...] = jnp.zeros_like(racc)

    @pl.when(p == 0)
    def _():
        t = (dinv_ref[...] * (p0_ref[...] + p1_ref[...] + hs_ref[...])
             + bconv_ref[...])
        tbuf[pl.ds(i * R, R), :] = t
        acc[0:1, :] += jnp.sum(t, axis=0, keepdims=True)
        acc[1:2, :] += jnp.sum(t * t, axis=0, keepdims=True)

    @pl.when(p == 1)
    def _():
        mean = acc[0:1, :] / N
        msq = acc[1:2, :] / N
        var = msq - mean * mean
        inv = lax.rsqrt(var + 1e-5)
        scale = gamma_ref[...] * inv
        shift = beta_ref[...] - mean * scale

        ha = jnp.maximum(tbuf[pl.ds(i * R, R), :] * scale + shift, 0.0)
        gids = lax.broadcasted_iota(jnp.int32, (R, G), 1)
        onehot = jnp.where(batch_ref[...] == gids, 1.0, 0.0)
        racc[...] += lax.dot_general(onehot, ha, (((0,), (0,)), ((), ())),
                                     preferred_element_type=jnp.float32)

        @pl.when(i == NB - 1)
        def _():
            out_ref[...] = (jnp.dot(racc[...], wpost_ref[...],
                                    preferred_element_type=jnp.float32)
                            + bpost_ref[...])


def _tc_tail(aggp, hs, dinv, b_conv, gamma, beta, batch2d,
             W_post_pad, b_post_pad):
    return pl.pallas_call(
        _tc_tail_body,
        grid=(2, NB),
        in_specs=[
            pl.BlockSpec((R, H), lambda p, i: ((1 - p) * i, 0)),
            pl.BlockSpec((R, H), lambda p, i: ((1 - p) * i + NB, 0)),
            pl.BlockSpec((R, H), lambda p, i: ((1 - p) * i, 0)),
            pl.BlockSpec((R, 1), lambda p, i: ((1 - p) * i, 0)),
            pl.BlockSpec((1, H), lambda p, i: (0, 0)),
            pl.BlockSpec((1, H), lambda p, i: (0, 0)),
            pl.BlockSpec((1, H), lambda p, i: (0, 0)),
            pl.BlockSpec((R, 1), lambda p, i: (p * i, 0)),
            pl.BlockSpec((H, 128), lambda p, i: (0, 0)),
            pl.BlockSpec((1, 128), lambda p, i: (0, 0)),
        ],
        out_specs=pl.BlockSpec((G, 128), lambda p, i: (0, 0)),
        out_shape=jax.ShapeDtypeStruct((G, 128), jnp.float32),
        scratch_shapes=[
            pltpu.VMEM((N, H), jnp.float32),
            pltpu.VMEM((2, H), jnp.float32),
            pltpu.VMEM((G, H), jnp.float32),
        ],
    )(aggp, aggp, hs, dinv, b_conv.reshape(1, H), gamma.reshape(1, H),
      beta.reshape(1, H), batch2d, W_post_pad, b_post_pad)


def kernel(x, edge_index, batch, W_pre, b_pre, W_conv, b_conv, gamma, beta,
           W_post, b_post):
    src = edge_index[0]
    dst = edge_index[1]

    hist = _sc_degree(dst)

    deg0 = hist[:N].reshape(N, 1)
    deg1 = hist[NP:NP + N].reshape(N, 1)
    hs, dinv = _tc_head(x, W_pre, b_pre, W_conv, deg0, deg1)

    aggp = _sc_edge_agg(src, dst, hs)

    W_post_pad = jnp.pad(W_post, ((0, 0), (0, 128 - C)))
    b_post_pad = jnp.pad(b_post, (0, 128 - C)).reshape(1, 128)
    out_pad = _tc_tail(aggp, hs, dinv, b_conv, gamma, beta,
                       batch.reshape(N, 1), W_post_pad, b_post_pad)
    return out_pad[:, :C]


# speedup vs baseline: 38.3129x; 1.2475x over previous
import functools

import jax
import jax.numpy as jnp
from jax import lax
from jax.experimental import pallas as pl
from jax.experimental.pallas import tpu as pltpu
from jax.experimental.pallas import tpu_sc as plsc

N = 10000
E = 320000
D = 128
H = 128
C = 40
G = 64

NW = 32
NP = 10240
RPT = NP // 16
K = 80
EPW = E // NW
NCH = EPW // K

R = 1000
NB = N // R

_mesh = plsc.VectorSubcoreMesh(core_axis_name="c", subcore_axis_name="s")


@functools.partial(
    pl.kernel,
    out_type=jax.ShapeDtypeStruct((2 * NP,), jnp.float32),
    mesh=_mesh,
    scratch_types=[
        pltpu.VMEM((EPW,), jnp.int32),
        pltpu.VMEM((K,), jnp.float32),
        pltpu.VMEM((RPT,), jnp.float32),
        pltpu.VMEM_SHARED((NP,), jnp.float32),
        pltpu.SemaphoreType.DMA,
    ],
)
def _sc_degree(dst_hbm, out_hbm, di, ones_v, zb, hist, sidx):
    cid = lax.axis_index("c")
    sid = lax.axis_index("s")
    wid = sid * 2 + cid
    base_w = pl.multiple_of(wid * EPW, 8)

    cidx = pltpu.async_copy(dst_hbm.at[pl.ds(base_w, EPW)], di, sidx)

    z16 = jnp.zeros((16,), jnp.float32)
    o16 = jnp.ones((16,), jnp.float32)

    def zb_body(j, carry):
        zb[pl.ds(j * 16, 16)] = z16
        return carry

    lax.fori_loop(0, RPT // 16, zb_body, 0)
    for j in range(K // 16):
        ones_v[pl.ds(j * 16, 16)] = o16

    pltpu.sync_copy(zb, hist.at[pl.ds(sid * RPT, RPT)])
    cidx.wait()
    plsc.subcore_barrier()

    def body(i, carry):
        off = pl.multiple_of(i * K, 8)
        pltpu.sync_copy(ones_v, hist.at[di.at[pl.ds(off, K)]], add=True)
        return carry

    lax.fori_loop(0, NCH, body, 0)

    plsc.subcore_barrier()
    off = pl.multiple_of(cid * NP + sid * RPT, 8)
    pltpu.sync_copy(hist.at[pl.ds(sid * RPT, RPT)], out_hbm.at[pl.ds(off, RPT)])


OSPAN = 624
ZR = 32


@functools.partial(
    pl.kernel,
    out_type=jax.ShapeDtypeStruct((2 * N, H), jnp.float32),
    mesh=_mesh,
    scratch_types=[
        pltpu.VMEM((EPW,), jnp.int32),
        pltpu.VMEM((EPW,), jnp.int32),
        pltpu.VMEM((K, H), jnp.float32),
        pltpu.VMEM((K, H), jnp.float32),
        pltpu.VMEM((ZR, H), jnp.float32),
        pltpu.VMEM_SHARED((NP, H), jnp.float32),
        pltpu.SemaphoreType.DMA,
        pltpu.SemaphoreType.DMA,
        pltpu.SemaphoreType.DMA,
    ],
)
def _sc_edge_agg(src_hbm, dst_hbm, hs_hbm, out_hbm, si, di, ra, rb, zbuf,
                 acc, sga, sgb, sidx):
    cid = lax.axis_index("c")
    sid = lax.axis_index("s")
    wid = sid * 2 + cid
    base_w = pl.multiple_of(wid * EPW, 8)

    ci_a = pltpu.async_copy(src_hbm.at[pl.ds(base_w, EPW)], si, sidx)
    ci_b = pltpu.async_copy(dst_hbm.at[pl.ds(base_w, EPW)], di, sidx)

    z16 = jnp.zeros((16,), jnp.float32)

    def zb_body(j, carry):
        r = j // (H // 16)
        c = j - r * (H // 16)
        zbuf[r, pl.ds(c * 16, 16)] = z16
        return carry

    lax.fori_loop(0, ZR * (H // 16), zb_body, 0)

    for j in range(RPT // ZR):
        pltpu.sync_copy(zbuf, acc.at[pl.ds(sid * RPT + j * ZR, ZR)])
    ci_a.wait()
    ci_b.wait()
    plsc.subcore_barrier()

    def start_chunk(i, rv, sem):
        off = pl.multiple_of(i * K, 8)
        pltpu.async_copy(hs_hbm.at[si.at[pl.ds(off, K)]], rv, sem)

    def fin_chunk(i, rv, sem):
        pltpu.make_async_copy(hs_hbm.at[si.at[pl.ds(0, K)]], rv, sem).wait()
        off = pl.multiple_of(i * K, 8)
        pltpu.sync_copy(rv, acc.at[di.at[pl.ds(off, K)]], add=True)

    start_chunk(0, ra, sga)

    def body(p, carry):
        a = p * 2
        b = a + 1

        @pl.when(b < NCH)
        def _():
            start_chunk(b, rb, sgb)

        fin_chunk(a, ra, sga)

        @pl.when(b < NCH)
        def _():
            @pl.when(b + 1 < NCH)
            def _():
                start_chunk(b + 1, ra, sga)

            fin_chunk(b, rb, sgb)

        return carry

    lax.fori_loop(0, (NCH + 1) // 2, body, 0)

    plsc.subcore_barrier()

    @pl.when(sid < 15)
    def _():
        pltpu.sync_copy(acc.at[pl.ds(sid * OSPAN, OSPAN)],
                        out_hbm.at[pl.ds(cid * N + sid * OSPAN, OSPAN)])

    @pl.when(sid == 15)
    def _():
        pltpu.sync_copy(acc.at[pl.ds(15 * OSPAN, N - 15 * OSPAN)],
                        out_hbm.at[pl.ds(cid * N + 15 * OSPAN, N - 15 * OSPAN)])


def _tc_head_body(x_ref, wpre_ref, wconv_ref, bpre_ref, d0_ref, d1_ref,
                  hs_ref, dinv_ref, wb, bb):
    i = pl.program_id(0)

    @pl.when(i == 0)
    def _():
        wb[...] = jnp.dot(wpre_ref[...], wconv_ref[...],
                          preferred_element_type=jnp.float32)
        bb[...] = jnp.dot(bpre_ref[...], wconv_ref[...],
                          preferred_element_type=jnp.float32)

    h2 = jnp.dot(x_ref[...], wb[...],
                 preferred_element_type=jnp.float32) + bb[...]
    deg = d0_ref[...] + d1_ref[...] + 1.0
    dinv = lax.rsqrt(jnp.maximum(deg, 1e-12))
    dinv_ref[...] = dinv
    hs_ref[...] = h2 * dinv


def _tc_head(x, W_pre, b_pre, W_conv, deg0, deg1):
    return pl.pallas_call(
        _tc_head_body,
        grid=(NB,),
        in_specs=[
            pl.BlockSpec((R, D), lambda i: (i, 0)),
            pl.BlockSpec((D, H), lambda i: (0, 0)),
            pl.BlockSpec((D, H), lambda i: (0, 0)),
            pl.BlockSpec((1, D), lambda i: (0, 0)),
            pl.BlockSpec((R, 1), lambda i: (i, 0)),
            pl.BlockSpec((R, 1), lambda i: (i, 0)),
        ],
        out_specs=[
            pl.BlockSpec((R, H), lambda i: (i, 0)),
            pl.BlockSpec((R, 1), lambda i: (i, 0)),
        ],
        out_shape=[
            jax.ShapeDtypeStruct((N, H), jnp.float32),
            jax.ShapeDtypeStruct((N, 1), jnp.float32),
        ],
        scratch_shapes=[
            pltpu.VMEM((D, H), jnp.float32),
            pltpu.VMEM((1, H), jnp.float32),
        ],
    )(x, W_pre, W_conv, b_pre.reshape(1, D), deg0, deg1)


def _tc_tail_body(p0_ref, p1_ref, hs_ref, dinv_ref, bconv_ref, gamma_ref,
                  beta_ref, batch_ref, wpost_ref, bpost_ref, out_ref,
                  tbuf, acc, racc):
    p = pl.program_id(0)
    i = pl.program_id(1)

    @pl.when((p == 0) & (i == 0))
    def _():
        acc[...] = jnp.zeros_like(acc)
        racc[...] = jnp.zeros_like(racc)

    @pl.when(p == 0)
    def _():
        t = (dinv_ref[...] * (p0_ref[...] + p1_ref[...] + hs_ref[...])
             + bconv_ref[...])
        tbuf[pl.ds(i * R, R), :] = t
        acc[0:1, :] += jnp.sum(t, axis=0, keepdims=True)
        acc[1:2, :] += jnp.sum(t * t, axis=0, keepdims=True)

    @pl.when(p == 1)
    def _():
        mean = acc[0:1, :] / N
        msq = acc[1:2, :] / N
        var = msq - mean * mean
        inv = lax.rsqrt(var + 1e-5)
        scale = gamma_ref[...] * inv
        shift = beta_ref[...] - mean * scale

        ha = jnp.maximum(tbuf[pl.ds(i * R, R), :] * scale + shift, 0.0)
        gids = lax.broadcasted_iota(jnp.int32, (R, G), 1)
        onehot = jnp.where(batch_ref[...] == gids, 1.0, 0.0)
        racc[...] += lax.dot_general(onehot, ha, (((0,), (0,)), ((), ())),
                                     preferred_element_type=jnp.float32)

        @pl.when(i == NB - 1)
        def _():
            out_ref[...] = (jnp.dot(racc[...], wpost_ref[...],
                                    preferred_element_type=jnp.float32)
                            + bpost_ref[...])


def _tc_tail(aggp, hs, dinv, b_conv, gamma, beta, batch2d,
             W_post_pad, b_post_pad):
    return pl.pallas_call(
        _tc_tail_body,
        grid=(2, NB),
        in_specs=[
            pl.BlockSpec((R, H), lambda p, i: ((1 - p) * i, 0)),
            pl.BlockSpec((R, H), lambda p, i: ((1 - p) * i + NB, 0)),
            pl.BlockSpec((R, H), lambda p, i: ((1 - p) * i, 0)),
            pl.BlockSpec((R, 1), lambda p, i: ((1 - p) * i, 0)),
            pl.BlockSpec((1, H), lambda p, i: (0, 0)),
            pl.BlockSpec((1, H), lambda p, i: (0, 0)),
            pl.BlockSpec((1, H), lambda p, i: (0, 0)),
            pl.BlockSpec((R, 1), lambda p, i: (p * i, 0)),
            pl.BlockSpec((H, 128), lambda p, i: (0, 0)),
            pl.BlockSpec((1, 128), lambda p, i: (0, 0)),
        ],
        out_specs=pl.BlockSpec((G, 128), lambda p, i: (0, 0)),
        out_shape=jax.ShapeDtypeStruct((G, 128), jnp.float32),
        scratch_shapes=[
            pltpu.VMEM((N, H), jnp.float32),
            pltpu.VMEM((2, H), jnp.float32),
            pltpu.VMEM((G, H), jnp.float32),
        ],
    )(aggp, aggp, hs, dinv, b_conv.reshape(1, H), gamma.reshape(1, H),
      beta.reshape(1, H), batch2d, W_post_pad, b_post_pad)


def kernel(x, edge_index, batch, W_pre, b_pre, W_conv, b_conv, gamma, beta,
           W_post, b_post):
    src = edge_index[0]
    dst = edge_index[1]

    hist = _sc_degree(dst)

    deg0 = hist[:N].reshape(N, 1)
    deg1 = hist[NP:NP + N].reshape(N, 1)
    hs, dinv = _tc_head(x, W_pre, b_pre, W_conv, deg0, deg1)

    aggp = _sc_edge_agg(src, dst, hs)

    W_post_pad = jnp.pad(W_post, ((0, 0), (0, 128 - C)))
    b_post_pad = jnp.pad(b_post, (0, 128 - C)).reshape(1, 128)
    out_pad = _tc_tail(aggp, hs, dinv, b_conv, gamma, beta,
                       batch.reshape(N, 1), W_post_pad, b_post_pad)
    return out_pad[:, :C]


# speedup vs baseline: 38.3587x; 1.0012x over previous
import functools

import jax
import jax.numpy as jnp
from jax import lax
from jax.experimental import pallas as pl
from jax.experimental.pallas import tpu as pltpu
from jax.experimental.pallas import tpu_sc as plsc

N = 10000
E = 320000
D = 128
H = 128
C = 40
G = 64

NW = 32
NP = 10240
RPT = NP // 16
K = 80
EPW = E // NW
NCH = EPW // K

R = 1000
NB = N // R

_mesh = plsc.VectorSubcoreMesh(core_axis_name="c", subcore_axis_name="s")


@functools.partial(
    pl.kernel,
    out_type=jax.ShapeDtypeStruct((2 * NP,), jnp.float32),
    mesh=_mesh,
    scratch_types=[
        pltpu.VMEM((EPW,), jnp.int32),
        pltpu.VMEM((K,), jnp.float32),
        pltpu.VMEM((RPT,), jnp.float32),
        pltpu.VMEM_SHARED((NP,), jnp.float32),
        pltpu.SemaphoreType.DMA,
    ],
)
def _sc_degree(dst_hbm, out_hbm, di, ones_v, zb, hist, sidx):
    cid = lax.axis_index("c")
    sid = lax.axis_index("s")
    wid = sid * 2 + cid
    base_w = pl.multiple_of(wid * EPW, 8)

    cidx = pltpu.async_copy(dst_hbm.at[pl.ds(base_w, EPW)], di, sidx)

    z16 = jnp.zeros((16,), jnp.float32)
    o16 = jnp.ones((16,), jnp.float32)

    def zb_body(j, carry):
        zb[pl.ds(j * 16, 16)] = z16
        return carry

    lax.fori_loop(0, RPT // 16, zb_body, 0)
    for j in range(K // 16):
        ones_v[pl.ds(j * 16, 16)] = o16

    pltpu.sync_copy(zb, hist.at[pl.ds(sid * RPT, RPT)])
    cidx.wait()
    plsc.subcore_barrier()

    def body(i, carry):
        off = pl.multiple_of(i * K, 8)
        pltpu.sync_copy(ones_v, hist.at[di.at[pl.ds(off, K)]], add=True)
        return carry

    lax.fori_loop(0, NCH, body, 0)

    plsc.subcore_barrier()
    off = pl.multiple_of(cid * NP + sid * RPT, 8)
    pltpu.sync_copy(hist.at[pl.ds(sid * RPT, RPT)], out_hbm.at[pl.ds(off, RPT)])


OSPAN = 624
ZR = 32


@functools.partial(
    pl.kernel,
    out_type=jax.ShapeDtypeStruct((2 * N, H), jnp.float32),
    mesh=_mesh,
    scratch_types=[
        pltpu.VMEM((EPW,), jnp.int32),
        pltpu.VMEM((EPW,), jnp.int32),
        pltpu.VMEM((K, H), jnp.float32),
        pltpu.VMEM((K, H), jnp.float32),
        pltpu.VMEM((ZR, H), jnp.float32),
        pltpu.VMEM_SHARED((NP, H), jnp.float32),
        pltpu.SemaphoreType.DMA,
        pltpu.SemaphoreType.DMA,
        pltpu.SemaphoreType.DMA,
    ],
)
def _sc_edge_agg(src_hbm, dst_hbm, hs_hbm, out_hbm, si, di, ra, rb, zbuf,
                 acc, sga, sgb, sidx):
    cid = lax.axis_index("c")
    sid = lax.axis_index("s")
    wid = sid * 2 + cid
    base_w = pl.multiple_of(wid * EPW, 8)

    ci_a = pltpu.async_copy(src_hbm.at[pl.ds(base_w, EPW)], si, sidx)
    ci_b = pltpu.async_copy(dst_hbm.at[pl.ds(base_w, EPW)], di, sidx)

    z16 = jnp.zeros((16,), jnp.float32)

    def zb_body(j, carry):
        r = j // (H // 16)
        c = j - r * (H // 16)
        zbuf[r, pl.ds(c * 16, 16)] = z16
        return carry

    lax.fori_loop(0, ZR * (H // 16), zb_body, 0)

    for j in range(RPT // ZR):
        pltpu.sync_copy(zbuf, acc.at[pl.ds(sid * RPT + j * ZR, ZR)])
    ci_a.wait()
    ci_b.wait()
    plsc.subcore_barrier()

    def start_chunk(i, rv, sem):
        off = pl.multiple_of(i * K, 8)
        pltpu.async_copy(hs_hbm.at[si.at[pl.ds(off, K)]], rv, sem)

    def fin_chunk(i, rv, sem):
        pltpu.make_async_copy(hs_hbm.at[si.at[pl.ds(0, K)]], rv, sem).wait()
        off = pl.multiple_of(i * K, 8)
        pltpu.sync_copy(rv, acc.at[di.at[pl.ds(off, K)]], add=True)

    start_chunk(0, ra, sga)

    def body(p, carry):
        a = p * 2
        b = a + 1

        @pl.when(b < NCH)
        def _():
            start_chunk(b, rb, sgb)

        fin_chunk(a, ra, sga)

        @pl.when(b < NCH)
        def _():
            @pl.when(b + 1 < NCH)
            def _():
                start_chunk(b + 1, ra, sga)

            fin_chunk(b, rb, sgb)

        return carry

    lax.fori_loop(0, (NCH + 1) // 2, body, 0)

    plsc.subcore_barrier()

    @pl.when(sid < 15)
    def _():
        pltpu.sync_copy(acc.at[pl.ds(sid * OSPAN, OSPAN)],
                        out_hbm.at[pl.ds(cid * N + sid * OSPAN, OSPAN)])

    @pl.when(sid == 15)
    def _():
        pltpu.sync_copy(acc.at[pl.ds(15 * OSPAN, N - 15 * OSPAN)],
                        out_hbm.at[pl.ds(cid * N + 15 * OSPAN, N - 15 * OSPAN)])


def _tc_mm_body(x_ref, wpre_ref, wconv_ref, bpre_ref, h2_ref, wb, bb):
    i = pl.program_id(0)

    @pl.when(i == 0)
    def _():
        wb[...] = jnp.dot(wpre_ref[...], wconv_ref[...],
                          preferred_element_type=jnp.float32)
        bb[...] = jnp.dot(bpre_ref[...], wconv_ref[...],
                          preferred_element_type=jnp.float32)

    h2_ref[...] = jnp.dot(x_ref[...], wb[...],
                          preferred_element_type=jnp.float32) + bb[...]


def _tc_mm(x, W_pre, b_pre, W_conv):
    return pl.pallas_call(
        _tc_mm_body,
        grid=(NB,),
        in_specs=[
            pl.BlockSpec((R, D), lambda i: (i, 0)),
            pl.BlockSpec((D, H), lambda i: (0, 0)),
            pl.BlockSpec((D, H), lambda i: (0, 0)),
            pl.BlockSpec((1, D), lambda i: (0, 0)),
        ],
        out_specs=pl.BlockSpec((R, H), lambda i: (i, 0)),
        out_shape=jax.ShapeDtypeStruct((N, H), jnp.float32),
        scratch_shapes=[
            pltpu.VMEM((D, H), jnp.float32),
            pltpu.VMEM((1, H), jnp.float32),
        ],
    )(x, W_pre, W_conv, b_pre.reshape(1, D))


def _tc_scale_body(h2_ref, d0_ref, d1_ref, hs_ref, dinv_ref):
    deg = d0_ref[...] + d1_ref[...] + 1.0
    dinv = lax.rsqrt(jnp.maximum(deg, 1e-12))
    dinv_ref[...] = dinv
    hs_ref[...] = h2_ref[...] * dinv


def _tc_scale(h2, deg0, deg1):
    return pl.pallas_call(
        _tc_scale_body,
        grid=(NB,),
        in_specs=[
            pl.BlockSpec((R, H), lambda i: (i, 0)),
            pl.BlockSpec((R, 1), lambda i: (i, 0)),
            pl.BlockSpec((R, 1), lambda i: (i, 0)),
        ],
        out_specs=[
            pl.BlockSpec((R, H), lambda i: (i, 0)),
            pl.BlockSpec((R, 1), lambda i: (i, 0)),
        ],
        out_shape=[
            jax.ShapeDtypeStruct((N, H), jnp.float32),
            jax.ShapeDtypeStruct((N, 1), jnp.float32),
        ],
    )(h2, deg0, deg1)


def _tc_tail_body(p0_ref, p1_ref, hs_ref, dinv_ref, bconv_ref, gamma_ref,
                  beta_ref, batch_ref, wpost_ref, bpost_ref, out_ref,
                  tbuf, acc, racc):
    p = pl.program_id(0)
    i = pl.program_id(1)

    @pl.when((p == 0) & (i == 0))
    def _():
        acc[...] = jnp.zeros_like(acc)
        racc[...] = jnp.zeros_like(racc)

    @pl.when(p == 0)
    def _():
        t = (dinv_ref[...] * (p0_ref[...] + p1_ref[...] + hs_ref[...])
             + bconv_ref[...])
        tbuf[pl.ds(i * R, R), :] = t
        acc[0:1, :] += jnp.sum(t, axis=0, keepdims=True)
        acc[1:2, :] += jnp.sum(t * t, axis=0, keepdims=True)

    @pl.when(p == 1)
    def _():
        mean = acc[0:1, :] / N
        msq = acc[1:2, :] / N
        var = msq - mean * mean
        inv = lax.rsqrt(var + 1e-5)
        scale = gamma_ref[...] * inv
        shift = beta_ref[...] - mean * scale

        ha = jnp.maximum(tbuf[pl.ds(i * R, R), :] * scale + shift, 0.0)
        gids = lax.broadcasted_iota(jnp.int32, (R, G), 1)
        onehot = jnp.where(batch_ref[...] == gids, 1.0, 0.0)
        racc[...] += lax.dot_general(onehot, ha, (((0,), (0,)), ((), ())),
                                     preferred_element_type=jnp.float32)

        @pl.when(i == NB - 1)
        def _():
            out_ref[...] = (jnp.dot(racc[...], wpost_ref[...],
                                    preferred_element_type=jnp.float32)
                            + bpost_ref[...])


def _tc_tail(aggp, hs, dinv, b_conv, gamma, beta, batch2d,
             W_post_pad, b_post_pad):
    return pl.pallas_call(
        _tc_tail_body,
        grid=(2, NB),
        in_specs=[
            pl.BlockSpec((R, H), lambda p, i: ((1 - p) * i, 0)),
            pl.BlockSpec((R, H), lambda p, i: ((1 - p) * i + NB, 0)),
            pl.BlockSpec((R, H), lambda p, i: ((1 - p) * i, 0)),
            pl.BlockSpec((R, 1), lambda p, i: ((1 - p) * i, 0)),
            pl.BlockSpec((1, H), lambda p, i: (0, 0)),
            pl.BlockSpec((1, H), lambda p, i: (0, 0)),
            pl.BlockSpec((1, H), lambda p, i: (0, 0)),
            pl.BlockSpec((R, 1), lambda p, i: (p * i, 0)),
            pl.BlockSpec((H, 128), lambda p, i: (0, 0)),
            pl.BlockSpec((1, 128), lambda p, i: (0, 0)),
        ],
        out_specs=pl.BlockSpec((G, 128), lambda p, i: (0, 0)),
        out_shape=jax.ShapeDtypeStruct((G, 128), jnp.float32),
        scratch_shapes=[
            pltpu.VMEM((N, H), jnp.float32),
            pltpu.VMEM((2, H), jnp.float32),
            pltpu.VMEM((G, H), jnp.float32),
        ],
    )(aggp, aggp, hs, dinv, b_conv.reshape(1, H), gamma.reshape(1, H),
      beta.reshape(1, H), batch2d, W_post_pad, b_post_pad)


def kernel(x, edge_index, batch, W_pre, b_pre, W_conv, b_conv, gamma, beta,
           W_post, b_post):
    src = edge_index[0]
    dst = edge_index[1]

    hist = _sc_degree(dst)
    h2 = _tc_mm(x, W_pre, b_pre, W_conv)

    deg0 = hist[:N].reshape(N, 1)
    deg1 = hist[NP:NP + N].reshape(N, 1)
    hs, dinv = _tc_scale(h2, deg0, deg1)

    aggp = _sc_edge_agg(src, dst, hs)

    W_post_pad = jnp.pad(W_post, ((0, 0), (0, 128 - C)))
    b_post_pad = jnp.pad(b_post, (0, 128 - C)).reshape(1, 128)
    out_pad = _tc_tail(aggp, hs, dinv, b_conv, gamma, beta,
                       batch.reshape(N, 1), W_post_pad, b_post_pad)
    return out_pad[:, :C]


# speedup vs baseline: 39.1168x; 1.0198x over previous
import functools

import jax
import jax.numpy as jnp
from jax import lax
from jax.experimental import pallas as pl
from jax.experimental.pallas import tpu as pltpu
from jax.experimental.pallas import tpu_sc as plsc

N = 10000
E = 320000
D = 128
H = 128
C = 40
G = 64

NW = 32
NP = 10240
RPT = NP // 16
K = 80
EPW = E // NW
NCH = EPW // K

R = 1000
NB = N // R

_mesh = plsc.VectorSubcoreMesh(core_axis_name="c", subcore_axis_name="s")


@functools.partial(
    pl.kernel,
    out_type=jax.ShapeDtypeStruct((2 * NP,), jnp.float32),
    mesh=_mesh,
    scratch_types=[
        pltpu.VMEM((EPW,), jnp.int32),
        pltpu.VMEM((K,), jnp.float32),
        pltpu.VMEM((RPT,), jnp.float32),
        pltpu.VMEM_SHARED((NP,), jnp.float32),
        pltpu.SemaphoreType.DMA,
    ],
)
def _sc_degree(dst_hbm, out_hbm, di, ones_v, zb, hist, sidx):
    cid = lax.axis_index("c")
    sid = lax.axis_index("s")
    wid = sid * 2 + cid
    base_w = pl.multiple_of(wid * EPW, 8)

    cidx = pltpu.async_copy(dst_hbm.at[pl.ds(base_w, EPW)], di, sidx)

    z16 = jnp.zeros((16,), jnp.float32)
    o16 = jnp.ones((16,), jnp.float32)

    def zb_body(j, carry):
        zb[pl.ds(j * 16, 16)] = z16
        return carry

    lax.fori_loop(0, RPT // 16, zb_body, 0)
    for j in range(K // 16):
        ones_v[pl.ds(j * 16, 16)] = o16

    pltpu.sync_copy(zb, hist.at[pl.ds(sid * RPT, RPT)])
    cidx.wait()
    plsc.subcore_barrier()

    def body(i, carry):
        off = pl.multiple_of(i * K, 8)
        pltpu.sync_copy(ones_v, hist.at[di.at[pl.ds(off, K)]], add=True)
        return carry

    lax.fori_loop(0, NCH, body, 0)

    plsc.subcore_barrier()
    off = pl.multiple_of(cid * NP + sid * RPT, 8)
    pltpu.sync_copy(hist.at[pl.ds(sid * RPT, RPT)], out_hbm.at[pl.ds(off, RPT)])


OSPAN = 624
ZR = 32


@functools.partial(
    pl.kernel,
    out_type=jax.ShapeDtypeStruct((2 * N, H), jnp.float32),
    mesh=_mesh,
    scratch_types=[
        pltpu.VMEM((EPW,), jnp.int32),
        pltpu.VMEM((EPW,), jnp.int32),
        pltpu.VMEM((K, H), jnp.float32),
        pltpu.VMEM((K, H), jnp.float32),
        pltpu.VMEM((ZR, H), jnp.float32),
        pltpu.VMEM_SHARED((NP, H), jnp.float32),
        pltpu.SemaphoreType.DMA,
        pltpu.SemaphoreType.DMA,
        pltpu.SemaphoreType.DMA,
    ],
)
def _sc_edge_agg(src_hbm, dst_hbm, hs_hbm, out_hbm, si, di, ra, rb, zbuf,
                 acc, sga, sgb, sidx):
    cid = lax.axis_index("c")
    sid = lax.axis_index("s")
    wid = sid * 2 + cid
    base_w = pl.multiple_of(wid * EPW, 8)

    ci_a = pltpu.async_copy(src_hbm.at[pl.ds(base_w, EPW)], si, sidx)
    ci_b = pltpu.async_copy(dst_hbm.at[pl.ds(base_w, EPW)], di, sidx)

    z16 = jnp.zeros((16,), jnp.float32)

    def zb_body(j, carry):
        r = j // (H // 16)
        c = j - r * (H // 16)
        zbuf[r, pl.ds(c * 16, 16)] = z16
        return carry

    lax.fori_loop(0, ZR * (H // 16), zb_body, 0)

    for j in range(RPT // ZR):
        pltpu.sync_copy(zbuf, acc.at[pl.ds(sid * RPT + j * ZR, ZR)])
    ci_a.wait()
    ci_b.wait()
    plsc.subcore_barrier()

    def start_chunk(i, rv, sem):
        off = pl.multiple_of(i * K, 8)
        pltpu.async_copy(hs_hbm.at[si.at[pl.ds(off, K)]], rv, sem)

    def fin_chunk(i, rv, sem):
        pltpu.make_async_copy(hs_hbm.at[si.at[pl.ds(0, K)]], rv, sem).wait()
        off = pl.multiple_of(i * K, 8)
        pltpu.sync_copy(rv, acc.at[di.at[pl.ds(off, K)]], add=True)

    start_chunk(0, ra, sga)

    def body(p, carry):
        a = p * 2
        b = a + 1

        @pl.when(b < NCH)
        def _():
            start_chunk(b, rb, sgb)

        fin_chunk(a, ra, sga)

        @pl.when(b < NCH)
        def _():
            @pl.when(b + 1 < NCH)
            def _():
                start_chunk(b + 1, ra, sga)

            fin_chunk(b, rb, sgb)

        return carry

    lax.fori_loop(0, (NCH + 1) // 2, body, 0)

    plsc.subcore_barrier()

    @pl.when(sid < 15)
    def _():
        pltpu.sync_copy(acc.at[pl.ds(sid * OSPAN, OSPAN)],
                        out_hbm.at[pl.ds(cid * N + sid * OSPAN, OSPAN)])

    @pl.when(sid == 15)
    def _():
        pltpu.sync_copy(acc.at[pl.ds(15 * OSPAN, N - 15 * OSPAN)],
                        out_hbm.at[pl.ds(cid * N + 15 * OSPAN, N - 15 * OSPAN)])


def _tc_mm_body(x_ref, wpre_ref, wconv_ref, bpre_ref, h2_ref, wb, bb):
    i = pl.program_id(0)

    @pl.when(i == 0)
    def _():
        wb[...] = jnp.dot(wpre_ref[...], wconv_ref[...],
                          preferred_element_type=jnp.float32)
        bb[...] = jnp.dot(bpre_ref[...], wconv_ref[...],
                          preferred_element_type=jnp.float32)

    h2_ref[...] = jnp.dot(x_ref[...], wb[...],
                          preferred_element_type=jnp.float32) + bb[...]


def _tc_mm(x, W_pre, b_pre, W_conv):
    return pl.pallas_call(
        _tc_mm_body,
        grid=(NB,),
        in_specs=[
            pl.BlockSpec((R, D), lambda i: (i, 0)),
            pl.BlockSpec((D, H), lambda i: (0, 0)),
            pl.BlockSpec((D, H), lambda i: (0, 0)),
            pl.BlockSpec((1, D), lambda i: (0, 0)),
        ],
        out_specs=pl.BlockSpec((R, H), lambda i: (i, 0)),
        out_shape=jax.ShapeDtypeStruct((N, H), jnp.float32),
        scratch_shapes=[
            pltpu.VMEM((D, H), jnp.float32),
            pltpu.VMEM((1, H), jnp.float32),
        ],
    )(x, W_pre, W_conv, b_pre.reshape(1, D))


def _tc_scale_body(h2_ref, hist_ref, hs_ref, dinv_ref):
    i = pl.program_id(0)
    d0 = hist_ref[pl.ds(i * R, R), :]
    d1 = hist_ref[pl.ds(NP + i * R, R), :]
    deg = d0 + d1 + 1.0
    dinv = lax.rsqrt(jnp.maximum(deg, 1e-12))
    dinv_ref[...] = dinv
    hs_ref[...] = h2_ref[...] * dinv


def _tc_scale(h2, hist2d):
    return pl.pallas_call(
        _tc_scale_body,
        grid=(NB,),
        in_specs=[
            pl.BlockSpec((R, H), lambda i: (i, 0)),
            pl.BlockSpec((2 * NP, 1), lambda i: (0, 0)),
        ],
        out_specs=[
            pl.BlockSpec((R, H), lambda i: (i, 0)),
            pl.BlockSpec((R, 1), lambda i: (i, 0)),
        ],
        out_shape=[
            jax.ShapeDtypeStruct((N, H), jnp.float32),
            jax.ShapeDtypeStruct((N, 1), jnp.float32),
        ],
    )(h2, hist2d)


def _tc_tail_body(p0_ref, p1_ref, hs_ref, dinv_ref, bconv_ref, gamma_ref,
                  beta_ref, batch_ref, wpost_ref, bpost_ref, out_ref,
                  tbuf, acc, racc):
    p = pl.program_id(0)
    i = pl.program_id(1)

    @pl.when((p == 0) & (i == 0))
    def _():
        acc[...] = jnp.zeros_like(acc)
        racc[...] = jnp.zeros_like(racc)

    @pl.when(p == 0)
    def _():
        t = (dinv_ref[...] * (p0_ref[...] + p1_ref[...] + hs_ref[...])
             + bconv_ref[...])
        tbuf[pl.ds(i * R, R), :] = t
        acc[0:1, :] += jnp.sum(t, axis=0, keepdims=True)
        acc[1:2, :] += jnp.sum(t * t, axis=0, keepdims=True)

    @pl.when(p == 1)
    def _():
        mean = acc[0:1, :] / N
        msq = acc[1:2, :] / N
        var = msq - mean * mean
        inv = lax.rsqrt(var + 1e-5)
        scale = gamma_ref[...] * inv
        shift = beta_ref[...] - mean * scale

        ha = jnp.maximum(tbuf[pl.ds(i * R, R), :] * scale + shift, 0.0)
        gids = lax.broadcasted_iota(jnp.int32, (R, G), 1)
        onehot = jnp.where(batch_ref[...] == gids, 1.0, 0.0)
        racc[...] += lax.dot_general(onehot, ha, (((0,), (0,)), ((), ())),
                                     preferred_element_type=jnp.float32)

        @pl.when(i == NB - 1)
        def _():
            out_ref[...] = (jnp.dot(racc[...], wpost_ref[...],
                                    preferred_element_type=jnp.float32)
                            + bpost_ref[...])


def _tc_tail(aggp, hs, dinv, b_conv, gamma, beta, batch2d, W_post, b_post2d):
    return pl.pallas_call(
        _tc_tail_body,
        grid=(2, NB),
        in_specs=[
            pl.BlockSpec((R, H), lambda p, i: ((1 - p) * i, 0)),
            pl.BlockSpec((R, H), lambda p, i: ((1 - p) * i + NB, 0)),
            pl.BlockSpec((R, H), lambda p, i: ((1 - p) * i, 0)),
            pl.BlockSpec((R, 1), lambda p, i: ((1 - p) * i, 0)),
            pl.BlockSpec((1, H), lambda p, i: (0, 0)),
            pl.BlockSpec((1, H), lambda p, i: (0, 0)),
            pl.BlockSpec((1, H), lambda p, i: (0, 0)),
            pl.BlockSpec((R, 1), lambda p, i: (p * i, 0)),
            pl.BlockSpec((H, C), lambda p, i: (0, 0)),
            pl.BlockSpec((1, C), lambda p, i: (0, 0)),
        ],
        out_specs=pl.BlockSpec((G, C), lambda p, i: (0, 0)),
        out_shape=jax.ShapeDtypeStruct((G, C), jnp.float32),
        scratch_shapes=[
            pltpu.VMEM((N, H), jnp.float32),
            pltpu.VMEM((2, H), jnp.float32),
            pltpu.VMEM((G, H), jnp.float32),
        ],
    )(aggp, aggp, hs, dinv, b_conv.reshape(1, H), gamma.reshape(1, H),
      beta.reshape(1, H), batch2d, W_post, b_post2d)


def kernel(x, edge_index, batch, W_pre, b_pre, W_conv, b_conv, gamma, beta,
           W_post, b_post):
    src = edge_index[0]
    dst = edge_index[1]

    hist = _sc_degree(dst)
    h2 = _tc_mm(x, W_pre, b_pre, W_conv)

    hs, dinv = _tc_scale(h2, hist.reshape(2 * NP, 1))

    aggp = _sc_edge_agg(src, dst, hs)

    return _tc_tail(aggp, hs, dinv, b_conv, gamma, beta,
                    batch.reshape(N, 1), W_post, b_post.reshape(1, C))


# speedup vs baseline: 39.6476x; 1.0136x over previous
import functools

import jax
import jax.numpy as jnp
from jax import lax
from jax.experimental import pallas as pl
from jax.experimental.pallas import tpu as pltpu
from jax.experimental.pallas import tpu_sc as plsc

N = 10000
E = 320000
D = 128
H = 128
C = 40
G = 64

NW = 32
NP = 10240
RPT = NP // 16
K = 80
EPW = E // NW
NCH = EPW // K

KH = 128
NCHH = EPW // KH
REMH = EPW - NCHH * KH

R = 1000
NB = N // R

_mesh = plsc.VectorSubcoreMesh(core_axis_name="c", subcore_axis_name="s")


@functools.partial(
    pl.kernel,
    out_type=jax.ShapeDtypeStruct((2 * NP,), jnp.float32),
    mesh=_mesh,
    scratch_types=[
        pltpu.VMEM((EPW,), jnp.int32),
        pltpu.VMEM((KH,), jnp.float32),
        pltpu.VMEM((RPT,), jnp.float32),
        pltpu.VMEM_SHARED((NP,), jnp.float32),
        pltpu.SemaphoreType.DMA,
    ],
)
def _sc_degree(dst_hbm, out_hbm, di, ones_v, zb, hist, sidx):
    cid = lax.axis_index("c")
    sid = lax.axis_index("s")
    wid = sid * 2 + cid
    base_w = pl.multiple_of(wid * EPW, 8)

    cidx = pltpu.async_copy(dst_hbm.at[pl.ds(base_w, EPW)], di, sidx)

    z16 = jnp.zeros((16,), jnp.float32)
    o16 = jnp.ones((16,), jnp.float32)

    def zb_body(j, carry):
        zb[pl.ds(j * 16, 16)] = z16
        return carry

    lax.fori_loop(0, RPT // 16, zb_body, 0)
    for j in range(KH // 16):
        ones_v[pl.ds(j * 16, 16)] = o16

    pltpu.sync_copy(zb, hist.at[pl.ds(sid * RPT, RPT)])
    cidx.wait()
    plsc.subcore_barrier()

    def body(i, carry):
        off = pl.multiple_of(i * KH, 8)
        pltpu.sync_copy(ones_v, hist.at[di.at[pl.ds(off, KH)]], add=True)
        return carry

    lax.fori_loop(0, NCHH, body, 0)

    @pl.when(REMH > 0)
    def _():
        pltpu.sync_copy(ones_v.at[pl.ds(0, REMH)],
                        hist.at[di.at[pl.ds(NCHH * KH, REMH)]], add=True)

    plsc.subcore_barrier()
    off = pl.multiple_of(cid * NP + sid * RPT, 8)
    pltpu.sync_copy(hist.at[pl.ds(sid * RPT, RPT)], out_hbm.at[pl.ds(off, RPT)])


OSPAN = 624
ZR = 32


@functools.partial(
    pl.kernel,
    out_type=jax.ShapeDtypeStruct((2 * N, H), jnp.float32),
    mesh=_mesh,
    scratch_types=[
        pltpu.VMEM((EPW,), jnp.int32),
        pltpu.VMEM((EPW,), jnp.int32),
        pltpu.VMEM((K, H), jnp.float32),
        pltpu.VMEM((K, H), jnp.float32),
        pltpu.VMEM((ZR, H), jnp.float32),
        pltpu.VMEM_SHARED((NP, H), jnp.float32),
        pltpu.SemaphoreType.DMA,
        pltpu.SemaphoreType.DMA,
        pltpu.SemaphoreType.DMA,
    ],
)
def _sc_edge_agg(src_hbm, dst_hbm, hs_hbm, out_hbm, si, di, ra, rb, zbuf,
                 acc, sga, sgb, sidx):
    cid = lax.axis_index("c")
    sid = lax.axis_index("s")
    wid = sid * 2 + cid
    base_w = pl.multiple_of(wid * EPW, 8)

    ci_a = pltpu.async_copy(src_hbm.at[pl.ds(base_w, EPW)], si, sidx)
    ci_b = pltpu.async_copy(dst_hbm.at[pl.ds(base_w, EPW)], di, sidx)

    z16 = jnp.zeros((16,), jnp.float32)

    def zb_body(j, carry):
        r = j // (H // 16)
        c = j - r * (H // 16)
        zbuf[r, pl.ds(c * 16, 16)] = z16
        return carry

    lax.fori_loop(0, ZR * (H // 16), zb_body, 0)

    for j in range(RPT // ZR):
        pltpu.sync_copy(zbuf, acc.at[pl.ds(sid * RPT + j * ZR, ZR)])
    ci_a.wait()
    ci_b.wait()
    plsc.subcore_barrier()

    def start_chunk(i, rv, sem):
        off = pl.multiple_of(i * K, 8)
        pltpu.async_copy(hs_hbm.at[si.at[pl.ds(off, K)]], rv, sem)

    def fin_chunk(i, rv, sem):
        pltpu.make_async_copy(hs_hbm.at[si.at[pl.ds(0, K)]], rv, sem).wait()
        off = pl.multiple_of(i * K, 8)
        pltpu.sync_copy(rv, acc.at[di.at[pl.ds(off, K)]], add=True)

    start_chunk(0, ra, sga)

    def body(p, carry):
        a = p * 2
        b = a + 1

        @pl.when(b < NCH)
        def _():
            start_chunk(b, rb, sgb)

        fin_chunk(a, ra, sga)

        @pl.when(b < NCH)
        def _():
            @pl.when(b + 1 < NCH)
            def _():
                start_chunk(b + 1, ra, sga)

            fin_chunk(b, rb, sgb)

        return carry

    lax.fori_loop(0, (NCH + 1) // 2, body, 0)

    plsc.subcore_barrier()

    @pl.when(sid < 15)
    def _():
        pltpu.sync_copy(acc.at[pl.ds(sid * OSPAN, OSPAN)],
                        out_hbm.at[pl.ds(cid * N + sid * OSPAN, OSPAN)])

    @pl.when(sid == 15)
    def _():
        pltpu.sync_copy(acc.at[pl.ds(15 * OSPAN, N - 15 * OSPAN)],
                        out_hbm.at[pl.ds(cid * N + 15 * OSPAN, N - 15 * OSPAN)])


def _tc_mm_body(x_ref, wpre_ref, wconv_ref, bpre_ref, h2_ref, wb, bb):
    i = pl.program_id(0)

    @pl.when(i == 0)
    def _():
        wb[...] = jnp.dot(wpre_ref[...], wconv_ref[...],
                          preferred_element_type=jnp.float32)
        bb[...] = jnp.dot(bpre_ref[...], wconv_ref[...],
                          preferred_element_type=jnp.float32)

    h2_ref[...] = jnp.dot(x_ref[...], wb[...],
                          preferred_element_type=jnp.float32) + bb[...]


def _tc_mm(x, W_pre, b_pre, W_conv):
    return pl.pallas_call(
        _tc_mm_body,
        grid=(NB,),
        in_specs=[
            pl.BlockSpec((R, D), lambda i: (i, 0)),
            pl.BlockSpec((D, H), lambda i: (0, 0)),
            pl.BlockSpec((D, H), lambda i: (0, 0)),
            pl.BlockSpec((1, D), lambda i: (0, 0)),
        ],
        out_specs=pl.BlockSpec((R, H), lambda i: (i, 0)),
        out_shape=jax.ShapeDtypeStruct((N, H), jnp.float32),
        scratch_shapes=[
            pltpu.VMEM((D, H), jnp.float32),
            pltpu.VMEM((1, H), jnp.float32),
        ],
    )(x, W_pre, W_conv, b_pre.reshape(1, D))


def _tc_scale_body(h2_ref, hist_ref, hs_ref, dinv_ref):
    i = pl.program_id(0)
    d0 = hist_ref[pl.ds(i * R, R), :]
    d1 = hist_ref[pl.ds(NP + i * R, R), :]
    deg = d0 + d1 + 1.0
    dinv = lax.rsqrt(jnp.maximum(deg, 1e-12))
    dinv_ref[...] = dinv
    hs_ref[...] = h2_ref[...] * dinv


def _tc_scale(h2, hist2d):
    return pl.pallas_call(
        _tc_scale_body,
        grid=(NB,),
        in_specs=[
            pl.BlockSpec((R, H), lambda i: (i, 0)),
            pl.BlockSpec((2 * NP, 1), lambda i: (0, 0)),
        ],
        out_specs=[
            pl.BlockSpec((R, H), lambda i: (i, 0)),
            pl.BlockSpec((R, 1), lambda i: (i, 0)),
        ],
        out_shape=[
            jax.ShapeDtypeStruct((N, H), jnp.float32),
            jax.ShapeDtypeStruct((N, 1), jnp.float32),
        ],
    )(h2, hist2d)


def _tc_tail_body(p0_ref, p1_ref, hs_ref, dinv_ref, bconv_ref, gamma_ref,
                  beta_ref, batch_ref, wpost_ref, bpost_ref, out_ref,
                  tbuf, acc, racc):
    p = pl.program_id(0)
    i = pl.program_id(1)

    @pl.when((p == 0) & (i == 0))
    def _():
        acc[...] = jnp.zeros_like(acc)
        racc[...] = jnp.zeros_like(racc)

    @pl.when(p == 0)
    def _():
        t = (dinv_ref[...] * (p0_ref[...] + p1_ref[...] + hs_ref[...])
             + bconv_ref[...])
        tbuf[pl.ds(i * R, R), :] = t
        acc[0:1, :] += jnp.sum(t, axis=0, keepdims=True)
        acc[1:2, :] += jnp.sum(t * t, axis=0, keepdims=True)

    @pl.when(p == 1)
    def _():
        mean = acc[0:1, :] / N
        msq = acc[1:2, :] / N
        var = msq - mean * mean
        inv = lax.rsqrt(var + 1e-5)
        scale = gamma_ref[...] * inv
        shift = beta_ref[...] - mean * scale

        ha = jnp.maximum(tbuf[pl.ds(i * R, R), :] * scale + shift, 0.0)
        gids = lax.broadcasted_iota(jnp.int32, (R, G), 1)
        onehot = jnp.where(batch_ref[...] == gids, 1.0, 0.0)
        racc[...] += lax.dot_general(onehot, ha, (((0,), (0,)), ((), ())),
                                     preferred_element_type=jnp.float32)

        @pl.when(i == NB - 1)
        def _():
            out_ref[...] = (jnp.dot(racc[...], wpost_ref[...],
                                    preferred_element_type=jnp.float32)
                            + bpost_ref[...])


def _tc_tail(aggp, hs, dinv, b_conv, gamma, beta, batch2d, W_post, b_post2d):
    return pl.pallas_call(
        _tc_tail_body,
        grid=(2, NB),
        in_specs=[
            pl.BlockSpec((R, H), lambda p, i: ((1 - p) * i, 0)),
            pl.BlockSpec((R, H), lambda p, i: ((1 - p) * i + NB, 0)),
            pl.BlockSpec((R, H), lambda p, i: ((1 - p) * i, 0)),
            pl.BlockSpec((R, 1), lambda p, i: ((1 - p) * i, 0)),
            pl.BlockSpec((1, H), lambda p, i: (0, 0)),
            pl.BlockSpec((1, H), lambda p, i: (0, 0)),
            pl.BlockSpec((1, H), lambda p, i: (0, 0)),
            pl.BlockSpec((R, 1), lambda p, i: (p * i, 0)),
            pl.BlockSpec((H, C), lambda p, i: (0, 0)),
            pl.BlockSpec((1, C), lambda p, i: (0, 0)),
        ],
        out_specs=pl.BlockSpec((G, C), lambda p, i: (0, 0)),
        out_shape=jax.ShapeDtypeStruct((G, C), jnp.float32),
        scratch_shapes=[
            pltpu.VMEM((N, H), jnp.float32),
            pltpu.VMEM((2, H), jnp.float32),
            pltpu.VMEM((G, H), jnp.float32),
        ],
    )(aggp, aggp, hs, dinv, b_conv.reshape(1, H), gamma.reshape(1, H),
      beta.reshape(1, H), batch2d, W_post, b_post2d)


def kernel(x, edge_index, batch, W_pre, b_pre, W_conv, b_conv, gamma, beta,
           W_post, b_post):
    src = edge_index[0]
    dst = edge_index[1]

    hist = _sc_degree(dst)
    h2 = _tc_mm(x, W_pre, b_pre, W_conv)

    hs, dinv = _tc_scale(h2, hist.reshape(2 * NP, 1))

    aggp = _sc_edge_agg(src, dst, hs)

    return _tc_tail(aggp, hs, dinv, b_conv, gamma, beta,
                    batch.reshape(N, 1), W_post, b_post.reshape(1, C))
